# baseline (speedup 1.0000x reference)
import functools

import jax
import jax.numpy as jnp
from jax import lax
from jax.experimental import pallas as pl
from jax.experimental.pallas import tpu as pltpu
from jax.experimental.pallas import tpu_sc as plsc

_C = 4.0
_TAU = 1.0

_NC = 1
_NS = 16
_LANES = 16
_CR = 64


def _bf16_round(v):
    u = lax.bitcast_convert_type(v, jnp.int32)
    r = u + (jnp.int32(0x7FFF) + ((u >> 16) & 1))
    return lax.bitcast_convert_type(r & jnp.int32(-65536), jnp.float32)


def _sc_body(L, rows_per_worker, sh_slice, x_ref, s_ref, params_ref,
             logit_ref, label_ref, idx_a, idx_b, vals_a, vals_b,
             params_v, lg_v, lb_v, s_sh, sem_ga, sem_gb, sem_ia, sem_ib):
    wid = lax.axis_index("s") * _NC + lax.axis_index("c")
    sid = lax.axis_index("s")
    n_chunks = rows_per_worker // _CR
    n_groups = _CR // _LANES
    cl_elems = _CR * L

    ibuf = [idx_a, idx_b]
    vbuf = [vals_a, vals_b]
    gsem = [sem_ga, sem_gb]
    isem = [sem_ia, sem_ib]

    pltpu.sync_copy(params_ref, params_v)
    w0 = _bf16_round(params_v[0])
    w1 = _bf16_round(params_v[1])
    w2 = _bf16_round(params_v[2])
    bias = params_v[3]

    def idx_start(c):
        cb = wid * n_chunks + c
        return pltpu.async_copy(
            x_ref.at[pl.ds(cb * cl_elems, cl_elems)], ibuf[c & 1], isem[c & 1])

    def gather_start(c):
        return pltpu.async_copy(s_sh.at[ibuf[c & 1]], vbuf[c & 1], gsem[c & 1])

    del sh_slice

    @pl.when(sid == 0)
    def _stage():
        pltpu.sync_copy(s_ref, s_sh)

    plsc.subcore_barrier()

    idx_start(0).wait()
    pend_g = gather_start(0)
    pend_i = idx_start(1) if n_chunks > 1 else None

    for c in range(n_chunks):
        cb = wid * n_chunks + c
        pend_g.wait()
        if c + 1 < n_chunks:
            pend_i.wait()
            pend_g = gather_start(c + 1)
        if c + 2 < n_chunks:
            pend_i = idx_start(c + 2)

        vals_v = vbuf[c & 1]
        for g in range(n_groups):
            goff = g * _LANES

            def pos_body(j, sums, goff=goff, vals_v=vals_v):
                s_cl, s_p, s_n = sums
                v = vals_v[pl.ds(j * _CR + goff, _LANES)]
                cl = jnp.minimum(jnp.maximum(v, -_C), _C)
                s_cl = s_cl + _bf16_round(cl)
                s_p = s_p + jnp.where(cl > _TAU, 1.0, 0.0)
                s_n = s_n + jnp.where(cl < -_TAU, 1.0, 0.0)
                return s_cl, s_p, s_n

            zeros = jnp.zeros((_LANES,), jnp.float32)
            s_cl, s_p, s_n = lax.fori_loop(
                0, L, pos_body, (zeros, zeros, zeros), unroll=4)

            logit = _bf16_round(s_cl) * w0 + s_p * w1 + s_n * w2 + bias
            ones_i = jnp.full((_LANES,), 1, jnp.int32)
            zeros_i = jnp.zeros((_LANES,), jnp.int32)
            label = jnp.where(logit >= 0.0, ones_i, zeros_i)
            lg_v[pl.ds(goff, _LANES)] = logit
            lb_v[pl.ds(goff, _LANES)] = label

        pltpu.sync_copy(lg_v, logit_ref.at[pl.ds(cb * _CR, _CR)])
        pltpu.sync_copy(lb_v, label_ref.at[pl.ds(cb * _CR, _CR)])


@functools.partial(jax.jit, static_argnums=(3, 4))
def _run(x_perm, s_flat, params, B, L):
    rows_per_worker = B // (_NC * _NS)
    v_pad = s_flat.shape[0]
    sh_slice = v_pad // _NS

    mesh = plsc.VectorSubcoreMesh(core_axis_name="c", subcore_axis_name="s",
                                  num_cores=_NC)
    fn = pl.kernel(
        functools.partial(_sc_body, L, rows_per_worker, sh_slice),
        out_type=(
            jax.ShapeDtypeStruct((B,), jnp.float32),
            jax.ShapeDtypeStruct((B,), jnp.int32),
        ),
        mesh=mesh,
        scratch_types=[
            pltpu.VMEM((_CR * L,), jnp.int32),
            pltpu.VMEM((_CR * L,), jnp.int32),
            pltpu.VMEM((_CR * L,), jnp.float32),
            pltpu.VMEM((_CR * L,), jnp.float32),
            pltpu.VMEM((4, _LANES), jnp.float32),
            pltpu.VMEM((_CR,), jnp.float32),
            pltpu.VMEM((_CR,), jnp.int32),
            pltpu.VMEM_SHARED((v_pad,), jnp.float32),
            pltpu.SemaphoreType.DMA,
            pltpu.SemaphoreType.DMA,
            pltpu.SemaphoreType.DMA,
            pltpu.SemaphoreType.DMA,
        ],
    )
    return fn(x_perm, s_flat, params)


def kernel(x, S, head_W, head_b):
    B, L = x.shape
    x_perm = x.reshape(B // _CR, _CR, L).swapaxes(1, 2).reshape(-1)
    s_flat = S.reshape(-1)
    v_pad = ((s_flat.shape[0] + (16 * 16) - 1) // (16 * 16)) * (16 * 16)
    s_flat = jnp.pad(s_flat, (0, v_pad - s_flat.shape[0]))
    splat = jnp.ones((_LANES,), jnp.float32)
    params = jnp.stack([
        splat * head_W[0, 0],
        splat * head_W[0, 1],
        splat * head_W[0, 2],
        splat * head_b[0],
    ])
    logit, label = _run(x_perm, s_flat, params, B, L)
    return logit.reshape(B, 1), label.reshape(B, 1)


# speedup vs baseline: 1.2042x; 1.2042x over previous
import functools

import jax
import jax.numpy as jnp
from jax import lax
from jax.experimental import pallas as pl
from jax.experimental.pallas import tpu as pltpu
from jax.experimental.pallas import tpu_sc as plsc

_C = 4.0
_TAU = 1.0

_NC = 2
_NS = 16
_LANES = 16
_CR = 64


def _bf16_round(v):
    u = lax.bitcast_convert_type(v, jnp.int32)
    r = u + (jnp.int32(0x7FFF) + ((u >> 16) & 1))
    return lax.bitcast_convert_type(r & jnp.int32(-65536), jnp.float32)


def _sc_body(L, rows_per_worker, sh_slice, x_ref, s_ref, params_ref,
             logit_ref, label_ref, idx_a, idx_b, vals_a, vals_b,
             params_v, lg_v, lb_v, s_sh, sem_ga, sem_gb, sem_ia, sem_ib):
    wid = lax.axis_index("s") * _NC + lax.axis_index("c")
    sid = lax.axis_index("s")
    n_chunks = rows_per_worker // _CR
    n_groups = _CR // _LANES
    cl_elems = _CR * L

    ibuf = [idx_a, idx_b]
    vbuf = [vals_a, vals_b]
    gsem = [sem_ga, sem_gb]
    isem = [sem_ia, sem_ib]

    pltpu.sync_copy(params_ref, params_v)
    w0 = _bf16_round(params_v[0])
    w1 = _bf16_round(params_v[1])
    w2 = _bf16_round(params_v[2])
    bias = params_v[3]

    def idx_start(c):
        cb = wid * n_chunks + c
        return pltpu.async_copy(
            x_ref.at[pl.ds(cb * cl_elems, cl_elems)], ibuf[c & 1], isem[c & 1])

    def gather_start(c):
        return pltpu.async_copy(s_sh.at[ibuf[c & 1]], vbuf[c & 1], gsem[c & 1])

    del sh_slice

    @pl.when(sid == 0)
    def _stage():
        pltpu.sync_copy(s_ref, s_sh)

    plsc.subcore_barrier()

    idx_start(0).wait()
    pend_g = gather_start(0)
    pend_i = idx_start(1) if n_chunks > 1 else None

    for c in range(n_chunks):
        cb = wid * n_chunks + c
        if c + 1 < n_chunks:
            pend_i.wait()
            next_g = gather_start(c + 1)
        pend_g.wait()
        if c + 1 < n_chunks:
            pend_g = next_g
        if c + 2 < n_chunks:
            pend_i = idx_start(c + 2)

        vals_v = vbuf[c & 1]
        for g in range(n_groups):
            goff = g * _LANES

            def pos_body(j, sums, goff=goff, vals_v=vals_v):
                s_cl, s_p, s_n = sums
                v = vals_v[pl.ds(j * _CR + goff, _LANES)]
                cl = jnp.minimum(jnp.maximum(v, -_C), _C)
                s_cl = s_cl + _bf16_round(cl)
                s_p = s_p + jnp.where(cl > _TAU, 1.0, 0.0)
                s_n = s_n + jnp.where(cl < -_TAU, 1.0, 0.0)
                return s_cl, s_p, s_n

            zeros = jnp.zeros((_LANES,), jnp.float32)
            s_cl, s_p, s_n = lax.fori_loop(
                0, L, pos_body, (zeros, zeros, zeros), unroll=4)

            logit = _bf16_round(s_cl) * w0 + s_p * w1 + s_n * w2 + bias
            ones_i = jnp.full((_LANES,), 1, jnp.int32)
            zeros_i = jnp.zeros((_LANES,), jnp.int32)
            label = jnp.where(logit >= 0.0, ones_i, zeros_i)
            lg_v[pl.ds(goff, _LANES)] = logit
            lb_v[pl.ds(goff, _LANES)] = label

        pltpu.sync_copy(lg_v, logit_ref.at[pl.ds(cb * _CR, _CR)])
        pltpu.sync_copy(lb_v, label_ref.at[pl.ds(cb * _CR, _CR)])


@functools.partial(jax.jit, static_argnums=(3, 4))
def _run(x_perm, s_flat, params, B, L):
    rows_per_worker = B // (_NC * _NS)
    v_pad = s_flat.shape[0]
    sh_slice = v_pad // _NS

    mesh = plsc.VectorSubcoreMesh(core_axis_name="c", subcore_axis_name="s")
    fn = pl.kernel(
        functools.partial(_sc_body, L, rows_per_worker, sh_slice),
        out_type=(
            jax.ShapeDtypeStruct((B,), jnp.float32),
            jax.ShapeDtypeStruct((B,), jnp.int32),
        ),
        mesh=mesh,
        scratch_types=[
            pltpu.VMEM((_CR * L,), jnp.int32),
            pltpu.VMEM((_CR * L,), jnp.int32),
            pltpu.VMEM((_CR * L,), jnp.float32),
            pltpu.VMEM((_CR * L,), jnp.float32),
            pltpu.VMEM((4, _LANES), jnp.float32),
            pltpu.VMEM((_CR,), jnp.float32),
            pltpu.VMEM((_CR,), jnp.int32),
            pltpu.VMEM_SHARED((v_pad,), jnp.float32),
            pltpu.SemaphoreType.DMA,
            pltpu.SemaphoreType.DMA,
            pltpu.SemaphoreType.DMA,
            pltpu.SemaphoreType.DMA,
        ],
    )
    return fn(x_perm, s_flat, params)


def kernel(x, S, head_W, head_b):
    B, L = x.shape
    x_perm = x.reshape(B // _CR, _CR, L).swapaxes(1, 2).reshape(-1)
    s_flat = S.reshape(-1)
    v_pad = ((s_flat.shape[0] + (16 * 16) - 1) // (16 * 16)) * (16 * 16)
    s_flat = jnp.pad(s_flat, (0, v_pad - s_flat.shape[0]))
    splat = jnp.ones((_LANES,), jnp.float32)
    params = jnp.stack([
        splat * head_W[0, 0],
        splat * head_W[0, 1],
        splat * head_W[0, 2],
        splat * head_b[0],
    ])
    logit, label = _run(x_perm, s_flat, params, B, L)
    return logit.reshape(B, 1), label.reshape(B, 1)


# speedup vs baseline: 1.2103x; 1.0051x over previous
import functools

import jax
import jax.numpy as jnp
from jax import lax
from jax.experimental import pallas as pl
from jax.experimental.pallas import tpu as pltpu
from jax.experimental.pallas import tpu_sc as plsc

_C = 4.0
_TAU = 1.0

_NC = 2
_NS = 16
_LANES = 16
_CR = 64


def _bf16_round(v):
    u = lax.bitcast_convert_type(v, jnp.int32)
    r = u + (jnp.int32(0x7FFF) + ((u >> 16) & 1))
    return lax.bitcast_convert_type(r & jnp.int32(-65536), jnp.float32)


def _sc_body(L, rows_per_worker, sh_slice, x_ref, s_ref, params_ref,
             logit_ref, label_ref, idx_a, idx_b, vals_a, vals_b,
             params_v, lg_v, lb_v, s_sh, sem_ga, sem_gb, sem_ia, sem_ib):
    wid = lax.axis_index("s") * _NC + lax.axis_index("c")
    sid = lax.axis_index("s")
    n_chunks = rows_per_worker // _CR
    n_groups = _CR // _LANES
    cl_elems = _CR * L

    ibuf = [idx_a, idx_b]
    vbuf = [vals_a, vals_b]
    gsem = [sem_ga, sem_gb]
    isem = [sem_ia, sem_ib]

    pltpu.sync_copy(params_ref, params_v)
    w0 = _bf16_round(params_v[0])
    w1 = _bf16_round(params_v[1])
    w2 = _bf16_round(params_v[2])
    bias = params_v[3]

    def idx_start(c):
        cb = wid * n_chunks + c
        return pltpu.async_copy(
            x_ref.at[pl.ds(cb * cl_elems, cl_elems)], ibuf[c & 1], isem[c & 1])

    def gather_start(c):
        return pltpu.async_copy(s_sh.at[ibuf[c & 1]], vbuf[c & 1], gsem[c & 1])

    del sh_slice

    @pl.when(sid == 0)
    def _stage():
        pltpu.sync_copy(s_ref, s_sh)

    plsc.subcore_barrier()

    idx_start(0).wait()
    pend_g = gather_start(0)
    pend_i = idx_start(1) if n_chunks > 1 else None

    for c in range(n_chunks):
        cb = wid * n_chunks + c
        pend_g.wait()
        if c + 1 < n_chunks:
            pend_i.wait()
            pend_g = gather_start(c + 1)
        if c + 2 < n_chunks:
            pend_i = idx_start(c + 2)

        vals_v = vbuf[c & 1]
        for g in range(n_groups):
            goff = g * _LANES

            def pos_body(j, sums, goff=goff, vals_v=vals_v):
                s_cl, s_p, s_n = sums
                v = vals_v[pl.ds(j * _CR + goff, _LANES)]
                cl = jnp.minimum(jnp.maximum(v, -_C), _C)
                s_cl = s_cl + _bf16_round(cl)
                s_p = s_p + jnp.where(cl > _TAU, 1.0, 0.0)
                s_n = s_n + jnp.where(cl < -_TAU, 1.0, 0.0)
                return s_cl, s_p, s_n

            zeros = jnp.zeros((_LANES,), jnp.float32)
            s_cl, s_p, s_n = lax.fori_loop(
                0, L, pos_body, (zeros, zeros, zeros), unroll=4)

            logit = _bf16_round(s_cl) * w0 + s_p * w1 + s_n * w2 + bias
            ones_i = jnp.full((_LANES,), 1, jnp.int32)
            zeros_i = jnp.zeros((_LANES,), jnp.int32)
            label = jnp.where(logit >= 0.0, ones_i, zeros_i)
            lg_v[pl.ds(goff, _LANES)] = logit
            lb_v[pl.ds(goff, _LANES)] = label

        pltpu.sync_copy(lg_v, logit_ref.at[pl.ds(cb * _CR, _CR)])
        pltpu.sync_copy(lb_v, label_ref.at[pl.ds(cb * _CR, _CR)])


@functools.partial(jax.jit, static_argnums=(3, 4))
def _run(x_perm, s_flat, params, B, L):
    rows_per_worker = B // (_NC * _NS)
    v_pad = s_flat.shape[0]
    sh_slice = v_pad // _NS

    mesh = plsc.VectorSubcoreMesh(core_axis_name="c", subcore_axis_name="s")
    fn = pl.kernel(
        functools.partial(_sc_body, L, rows_per_worker, sh_slice),
        out_type=(
            jax.ShapeDtypeStruct((B,), jnp.float32),
            jax.ShapeDtypeStruct((B,), jnp.int32),
        ),
        mesh=mesh,
        scratch_types=[
            pltpu.VMEM((_CR * L,), jnp.int32),
            pltpu.VMEM((_CR * L,), jnp.int32),
            pltpu.VMEM((_CR * L,), jnp.float32),
            pltpu.VMEM((_CR * L,), jnp.float32),
            pltpu.VMEM((4, _LANES), jnp.float32),
            pltpu.VMEM((_CR,), jnp.float32),
            pltpu.VMEM((_CR,), jnp.int32),
            pltpu.VMEM_SHARED((v_pad,), jnp.float32),
            pltpu.SemaphoreType.DMA,
            pltpu.SemaphoreType.DMA,
            pltpu.SemaphoreType.DMA,
            pltpu.SemaphoreType.DMA,
        ],
    )
    return fn(x_perm, s_flat, params)


def kernel(x, S, head_W, head_b):
    B, L = x.shape
    x_perm = x.reshape(B // _CR, _CR, L).swapaxes(1, 2).reshape(-1)
    s_flat = S.reshape(-1)
    splat = jnp.ones((_LANES,), jnp.float32)
    params = jnp.stack([
        splat * head_W[0, 0],
        splat * head_W[0, 1],
        splat * head_W[0, 2],
        splat * head_b[0],
    ])
    logit, label = _run(x_perm, s_flat, params, B, L)
    return logit.reshape(B, 1), label.reshape(B, 1)
